# baseline (speedup 1.0000x reference)
import functools

import jax
import jax.numpy as jnp
from jax import lax
from jax.experimental import pallas as pl
from jax.experimental.pallas import tpu as pltpu
from jax.experimental.pallas import tpu_sc as plsc

N = 10000
E = 320000
IN_FT = 128
OUT_FT = 256
NG = 64

NC = 2
NS = 16
NTILES = NC * NS
NPAD = 10240
RANGE = NPAD // NTILES
LIST_LEN = 12800
ECH = 2560
GCH = 64

_mesh = plsc.VectorSubcoreMesh(core_axis_name="c", subcore_axis_name="s")
_sc_params = pltpu.CompilerParams(needs_layout_passes=False)


def _filter_body(src_h, dst_h, ew_h, lsrc_h, ldst_h, lew_h,
                 in_src, in_dst, in_ew, ob_src, ob_dst, ob_ew):
    wid = lax.axis_index("s") * NC + lax.axis_index("c")
    lo = wid * RANGE
    iota = lax.broadcasted_iota(jnp.int32, (16,), 0)
    zi = jnp.zeros((16,), jnp.int32)
    zf = jnp.zeros((16,), jnp.float32)

    def prefill(i, _):
        b = i * 16
        ob_src[pl.ds(b, 16)] = (b + iota) & 8191
        ob_dst[pl.ds(b, 16)] = zi
        ob_ew[pl.ds(b, 16)] = zf
        return 0

    lax.fori_loop(0, LIST_LEN // 16, prefill, 0)

    def chunk(ci, off):
        pltpu.sync_copy(src_h.at[pl.ds(ci * ECH, ECH)], in_src)
        pltpu.sync_copy(dst_h.at[pl.ds(ci * ECH, ECH)], in_dst)
        pltpu.sync_copy(ew_h.at[pl.ds(ci * ECH, ECH)], in_ew)

        def inner(i, off):
            b = i * 16
            d = in_dst[pl.ds(b, 16)]
            m = (d >= lo) & (d < lo + RANGE)
            mi = m.astype(jnp.int32)
            cum = plsc.cumsum(mi)
            pos = off + cum - 1
            plsc.store_scatter(ob_src, [pos], in_src[pl.ds(b, 16)], mask=m)
            plsc.store_scatter(ob_dst, [pos], d - lo, mask=m)
            plsc.store_scatter(ob_ew, [pos], in_ew[pl.ds(b, 16)], mask=m)
            return off + jnp.sum(mi)

        return lax.fori_loop(0, ECH // 16, inner, off)

    lax.fori_loop(0, E // ECH, chunk, jnp.int32(0))

    pltpu.sync_copy(ob_src, lsrc_h.at[wid])
    pltpu.sync_copy(ob_dst, ldst_h.at[wid])
    pltpu.sync_copy(ob_ew, lew_h.at[wid])


_filter = functools.partial(
    pl.kernel,
    compiler_params=_sc_params,
    out_type=[jax.ShapeDtypeStruct((NTILES, LIST_LEN), jnp.int32),
              jax.ShapeDtypeStruct((NTILES, LIST_LEN), jnp.int32),
              jax.ShapeDtypeStruct((NTILES, LIST_LEN), jnp.float32)],
    mesh=_mesh,
    scratch_types=[pltpu.VMEM((ECH,), jnp.int32),
                   pltpu.VMEM((ECH,), jnp.int32),
                   pltpu.VMEM((ECH,), jnp.float32),
                   pltpu.VMEM((LIST_LEN,), jnp.int32),
                   pltpu.VMEM((LIST_LEN,), jnp.int32),
                   pltpu.VMEM((LIST_LEN,), jnp.float32)],
)(_filter_body)


def _make_agg(F):
    def body(x_h, lsrc_h, ldst_h, lew_h, agg_h,
             idx_v, dst_v, ew_v, rows_v, agg_v, sem):
        wid = lax.axis_index("s") * NC + lax.axis_index("c")
        iota = lax.broadcasted_iota(jnp.int32, (16,), 0)
        zf = jnp.zeros((16,), jnp.float32)

        def zero(i, _):
            agg_v[pl.ds(i * 16, 16)] = zf
            return 0

        lax.fori_loop(0, RANGE * F // 16, zero, 0)

        def chunk(ci, _):
            b = ci * GCH
            pltpu.sync_copy(lsrc_h.at[wid, pl.ds(b, GCH)], idx_v)
            pltpu.sync_copy(ldst_h.at[wid, pl.ds(b, GCH)], dst_v)
            pltpu.sync_copy(lew_h.at[wid, pl.ds(b, GCH)], ew_v)
            pltpu.async_copy(x_h.at[idx_v], rows_v, sem).wait()
            for g in range(GCH // 16):
                evec = g * 16 + iota
                dstv = dst_v[pl.ds(g * 16, 16)]
                eww = ew_v[pl.ds(g * 16, 16)]
                sbase = dstv * F

                def fblk(fb, _):
                    f0 = fb * 16
                    for k in range(16):
                        fvec = jnp.full((16,), f0 + k, jnp.int32)
                        vals = plsc.load_gather(rows_v, [evec, fvec])
                        plsc.addupdate_scatter(agg_v, [sbase + (f0 + k)],
                                               vals * eww)
                    return 0

                lax.fori_loop(0, F // 16, fblk, 0)
            return 0

        lax.fori_loop(0, LIST_LEN // GCH, chunk, 0)
        pltpu.sync_copy(agg_v, agg_h.at[wid])

    return functools.partial(
        pl.kernel,
        compiler_params=_sc_params,
        out_type=[jax.ShapeDtypeStruct((NTILES, RANGE * F), jnp.float32)],
        mesh=_mesh,
        scratch_types=[pltpu.VMEM((GCH,), jnp.int32),
                       pltpu.VMEM((GCH,), jnp.int32),
                       pltpu.VMEM((GCH,), jnp.float32),
                       pltpu.VMEM((GCH, F), jnp.float32),
                       pltpu.VMEM((RANGE * F,), jnp.float32),
                       pltpu.SemaphoreType.DMA],
    )(body)


_agg128 = _make_agg(IN_FT)
_agg256 = _make_agg(OUT_FT)

RB = 1024
_GRID = NPAD // RB
_PREC = lax.Precision.HIGHEST


def _prelu(t, a):
    return jnp.where(t >= 0, t, a * t)


def _onehot_pool(b_blk, h):
    oneh = (b_blk[:, :NG] ==
            lax.broadcasted_iota(jnp.int32, (b_blk.shape[0], NG), 1)
            ).astype(jnp.float32)
    return lax.dot_general(oneh, h, (((0,), (0,)), ((), ())),
                           preferred_element_type=jnp.float32,
                           precision=_PREC)


def _tc_first_body(x_ref, w0_ref, w1_ref, b_ref, a_ref, hw_ref, hg_ref):
    t = jnp.dot(x_ref[...], w0_ref[...], preferred_element_type=jnp.float32,
                precision=_PREC)
    h = _prelu(t, a_ref[0, 0])
    pg = _onehot_pool(b_ref[...], h)

    @pl.when(pl.program_id(0) == 0)
    def _():
        hg_ref[...] = jnp.zeros_like(hg_ref)

    hg_ref[...] += pg
    hw_ref[...] = jnp.dot(h, w1_ref[...], preferred_element_type=jnp.float32,
                          precision=_PREC)


def _tc_mid_body(x_ref, wn_ref, b_ref, a_ref, hw_ref, hg_ref):
    h = _prelu(x_ref[...], a_ref[0, 0])
    pg = _onehot_pool(b_ref[...], h)

    @pl.when(pl.program_id(0) == 0)
    def _():
        hg_ref[...] = jnp.zeros_like(hg_ref)

    hg_ref[...] += pg
    hw_ref[...] = jnp.dot(h, wn_ref[...], preferred_element_type=jnp.float32,
                          precision=_PREC)


def _tc_last_body(x_ref, b_ref, a_ref, h_ref, hg_ref):
    h = _prelu(x_ref[...], a_ref[0, 0])
    pg = _onehot_pool(b_ref[...], h)

    @pl.when(pl.program_id(0) == 0)
    def _():
        hg_ref[...] = jnp.zeros_like(hg_ref)

    hg_ref[...] += pg
    h_ref[...] = h


def _row_spec(cols):
    return pl.BlockSpec((RB, cols), lambda i: (i, 0))


def _fix_spec(shape):
    return pl.BlockSpec(shape, lambda i: (0, 0))


_tc_first = pl.pallas_call(
    _tc_first_body,
    grid=(_GRID,),
    in_specs=[_row_spec(IN_FT), _fix_spec((IN_FT, OUT_FT)),
              _fix_spec((OUT_FT, OUT_FT)), _row_spec(128),
              _fix_spec((1, 1))],
    out_specs=[_row_spec(OUT_FT), _fix_spec((NG, OUT_FT))],
    out_shape=[jax.ShapeDtypeStruct((NPAD, OUT_FT), jnp.float32),
               jax.ShapeDtypeStruct((NG, OUT_FT), jnp.float32)],
)

_tc_mid = pl.pallas_call(
    _tc_mid_body,
    grid=(_GRID,),
    in_specs=[_row_spec(OUT_FT), _fix_spec((OUT_FT, OUT_FT)),
              _row_spec(128), _fix_spec((1, 1))],
    out_specs=[_row_spec(OUT_FT), _fix_spec((NG, OUT_FT))],
    out_shape=[jax.ShapeDtypeStruct((NPAD, OUT_FT), jnp.float32),
               jax.ShapeDtypeStruct((NG, OUT_FT), jnp.float32)],
)

_tc_last = pl.pallas_call(
    _tc_last_body,
    grid=(_GRID,),
    in_specs=[_row_spec(OUT_FT), _row_spec(128), _fix_spec((1, 1))],
    out_specs=[_row_spec(OUT_FT), _fix_spec((NG, OUT_FT))],
    out_shape=[jax.ShapeDtypeStruct((NPAD, OUT_FT), jnp.float32),
               jax.ShapeDtypeStruct((NG, OUT_FT), jnp.float32)],
)


def kernel(feat, edge_index, batch_indices, edge_weight, W0, W1, W2,
           a0, a1, a2):
    feat_p = jnp.zeros((NPAD, IN_FT), jnp.float32).at[:N].set(feat)
    batch_p = jnp.full((NPAD,), NG, jnp.int32).at[:N].set(batch_indices)
    batch_b = jnp.broadcast_to(batch_p[:, None], (NPAD, 128))
    src = edge_index[0]
    dst = edge_index[1]

    lsrc, ldst, lew = _filter(src, dst, edge_weight)

    aggf = _agg128(feat_p, lsrc, ldst, lew)[0].reshape(NPAD, IN_FT)
    hw1, hg0 = _tc_first(aggf, W0, W1, batch_b, a0.reshape(1, 1))

    agg1 = _agg256(hw1, lsrc, ldst, lew)[0].reshape(NPAD, OUT_FT)
    hw2, hg1 = _tc_mid(agg1, W2, batch_b, a1.reshape(1, 1))

    agg2 = _agg256(hw2, lsrc, ldst, lew)[0].reshape(NPAD, OUT_FT)
    h3, hg2 = _tc_last(agg2, batch_b, a2.reshape(1, 1))

    return h3[:N], jnp.concatenate([hg0, hg1, hg2], axis=-1)


# speedup vs baseline: 10.6684x; 10.6684x over previous
import functools

import jax
import jax.numpy as jnp
from jax import lax
from jax.experimental import pallas as pl
from jax.experimental.pallas import tpu as pltpu
from jax.experimental.pallas import tpu_sc as plsc

N = 10000
E = 320000
IN_FT = 128
OUT_FT = 256
NG = 64

NC = 2
NS = 16
NTILES = NC * NS
NPAD = 10240
RANGE = NPAD // NTILES
LIST_LEN = 12800
ECH = 2560
GCH = 64
NCH = LIST_LEN // GCH
CWORDS = 3 * GCH
LWORDS = NCH * CWORDS

_mesh = plsc.VectorSubcoreMesh(core_axis_name="c", subcore_axis_name="s")
_sc_params = pltpu.CompilerParams(needs_layout_passes=False)


def _filter_body(src_h, dst_h, ew_h, lists_h,
                 in_src, in_dst, in_ew, ob):
    wid = lax.axis_index("s") * NC + lax.axis_index("c")
    lo = wid * RANGE
    iota = lax.broadcasted_iota(jnp.int32, (16,), 0)
    zi = jnp.zeros((16,), jnp.int32)

    def prefill(ci, _):
        base = ci * CWORDS
        for j in range(GCH // 16):
            ob[pl.ds(base + j * 16, 16)] = (base + j * 16 + iota) & 8191
            ob[pl.ds(base + GCH + j * 16, 16)] = zi
            ob[pl.ds(base + 2 * GCH + j * 16, 16)] = zi
        return 0

    lax.fori_loop(0, NCH, prefill, 0)

    def chunk(ci, off):
        pltpu.sync_copy(src_h.at[pl.ds(ci * ECH, ECH)], in_src)
        pltpu.sync_copy(dst_h.at[pl.ds(ci * ECH, ECH)], in_dst)
        pltpu.sync_copy(ew_h.at[pl.ds(ci * ECH, ECH)], in_ew)

        def inner(i, off):
            b = i * 16
            d = in_dst[pl.ds(b, 16)]
            m = (d >= lo) & (d < lo + RANGE)
            mi = m.astype(jnp.int32)
            cum = plsc.cumsum(mi)
            n = off + cum - 1
            pos = (lax.shift_right_logical(n, 6) * CWORDS) + (n & (GCH - 1))
            plsc.store_scatter(ob, [pos], in_src[pl.ds(b, 16)], mask=m)
            plsc.store_scatter(ob, [pos + GCH], d - lo, mask=m)
            plsc.store_scatter(ob, [pos + 2 * GCH],
                               plsc.bitcast(in_ew[pl.ds(b, 16)], jnp.int32),
                               mask=m)
            return off + jnp.sum(mi)

        return lax.fori_loop(0, ECH // 16, inner, off)

    lax.fori_loop(0, E // ECH, chunk, jnp.int32(0))

    obase = pl.multiple_of(wid * LWORDS, 128)
    pltpu.sync_copy(ob, lists_h.at[pl.ds(obase, LWORDS)])


_filter = functools.partial(
    pl.kernel,
    compiler_params=_sc_params,
    out_type=[jax.ShapeDtypeStruct((NTILES * LWORDS,), jnp.int32)],
    mesh=_mesh,
    scratch_types=[pltpu.VMEM((ECH,), jnp.int32),
                   pltpu.VMEM((ECH,), jnp.int32),
                   pltpu.VMEM((ECH,), jnp.float32),
                   pltpu.VMEM((LWORDS,), jnp.int32)],
)(_filter_body)


def _make_agg(F):
    FB = F // 16

    def body(x_h, lists_h, agg_h, lbuf, rows0, rows1, agg_v,
             sem_l, sem_g0, sem_g1):
        wid = lax.axis_index("s") * NC + lax.axis_index("c")
        iota = lax.broadcasted_iota(jnp.int32, (16,), 0)
        zf = jnp.zeros((16,), jnp.float32)

        def zero(i, _):
            agg_v[pl.ds(i * 16, 16)] = zf
            return 0

        lax.fori_loop(0, RANGE * F // 16, zero, 0)

        lbase = pl.multiple_of(wid * LWORDS, 128)

        pltpu.sync_copy(lists_h.at[pl.ds(lbase, CWORDS)],
                        lbuf.at[pl.ds(0, CWORDS)])
        pltpu.async_copy(x_h.at[lbuf.at[pl.ds(0, GCH)]], rows0, sem_g0)
        pltpu.async_copy(lists_h.at[pl.ds(lbase + CWORDS, CWORDS)],
                         lbuf.at[pl.ds(CWORDS, CWORDS)], sem_l)

        def substep(c, rows_cur, rows_nxt, sem_cur, sem_nxt):
            s_cur = pl.multiple_of((c % 3) * CWORDS, 8)
            s_n1 = pl.multiple_of(((c + 1) % 3) * CWORDS, 8)
            s_n2 = pl.multiple_of(((c + 2) % 3) * CWORDS, 8)

            @pl.when(c + 1 <= NCH - 1)
            def _():
                pltpu.make_async_copy(
                    lists_h.at[pl.ds(pl.multiple_of(lbase + (c + 1) * CWORDS, 8), CWORDS)],
                    lbuf.at[pl.ds(s_n1, CWORDS)], sem_l).wait()
                pltpu.async_copy(x_h.at[lbuf.at[pl.ds(s_n1, GCH)]],
                                 rows_nxt, sem_nxt)

            @pl.when(c + 2 <= NCH - 1)
            def _():
                pltpu.async_copy(
                    lists_h.at[pl.ds(pl.multiple_of(lbase + (c + 2) * CWORDS, 8), CWORDS)],
                    lbuf.at[pl.ds(s_n2, CWORDS)], sem_l)

            pltpu.make_async_copy(x_h.at[lbuf.at[pl.ds(s_cur, GCH)]],
                                  rows_cur, sem_cur).wait()

            def gblk(g, _):
                dvec = lbuf[pl.ds(s_cur + GCH + g * 16, 16)]
                wvec = plsc.bitcast(lbuf[pl.ds(s_cur + 2 * GCH + g * 16, 16)],
                                    jnp.float32)
                bvec = dvec * F
                e0 = g * 16
                bases = []
                ws = []
                for e16 in range(16):
                    sel = iota == e16
                    bases.append(jnp.sum(jnp.where(sel, bvec, 0)))
                    ws.append(jnp.sum(jnp.where(sel, wvec, 0.0)))
                for e16 in range(16):
                    base = bases[e16]
                    w = ws[e16]
                    e = e0 + e16
                    for fb0 in range(0, FB, 8):
                        nb = min(8, FB - fb0)
                        vals = [rows_cur[e, pl.ds((fb0 + j) * 16, 16)]
                                for j in range(nb)]
                        vals = [v * w for v in vals]
                        for j in range(nb):
                            plsc.addupdate(
                                agg_v.at[pl.ds(base + (fb0 + j) * 16, 16)],
                                vals[j])
                return 0

            lax.fori_loop(0, GCH // 16, gblk, 0)

        def loop(ci, _):
            substep(ci * 2, rows0, rows1, sem_g0, sem_g1)
            substep(ci * 2 + 1, rows1, rows0, sem_g1, sem_g0)
            return 0

        lax.fori_loop(0, NCH // 2, loop, 0)
        abase = pl.multiple_of(wid * (RANGE * F), 128)
        pltpu.sync_copy(agg_v, agg_h.at[pl.ds(abase, RANGE * F)])

    return functools.partial(
        pl.kernel,
        compiler_params=_sc_params,
        out_type=[jax.ShapeDtypeStruct((NTILES * RANGE * F,), jnp.float32)],
        mesh=_mesh,
        scratch_types=[pltpu.VMEM((3 * CWORDS,), jnp.int32),
                       pltpu.VMEM((GCH, F), jnp.float32),
                       pltpu.VMEM((GCH, F), jnp.float32),
                       pltpu.VMEM((RANGE * F,), jnp.float32),
                       pltpu.SemaphoreType.DMA,
                       pltpu.SemaphoreType.DMA,
                       pltpu.SemaphoreType.DMA],
    )(body)


_agg128 = _make_agg(IN_FT)
_agg256 = _make_agg(OUT_FT)

RB = 1024
_GRID = NPAD // RB
_PREC = lax.Precision.HIGHEST


def _prelu(t, a):
    return jnp.where(t >= 0, t, a * t)


def _onehot_pool(b_blk, h):
    oneh = (b_blk[:, :NG] ==
            lax.broadcasted_iota(jnp.int32, (b_blk.shape[0], NG), 1)
            ).astype(jnp.float32)
    return lax.dot_general(oneh, h, (((0,), (0,)), ((), ())),
                           preferred_element_type=jnp.float32,
                           precision=_PREC)


def _tc_first_body(x_ref, w0_ref, w1_ref, b_ref, a_ref, hw_ref, hg_ref):
    t = jnp.dot(x_ref[...], w0_ref[...], preferred_element_type=jnp.float32,
                precision=_PREC)
    h = _prelu(t, a_ref[0, 0])
    pg = _onehot_pool(b_ref[...], h)

    @pl.when(pl.program_id(0) == 0)
    def _():
        hg_ref[...] = jnp.zeros_like(hg_ref)

    hg_ref[...] += pg
    hw_ref[...] = jnp.dot(h, w1_ref[...], preferred_element_type=jnp.float32,
                          precision=_PREC)


def _tc_mid_body(x_ref, wn_ref, b_ref, a_ref, hw_ref, hg_ref):
    h = _prelu(x_ref[...], a_ref[0, 0])
    pg = _onehot_pool(b_ref[...], h)

    @pl.when(pl.program_id(0) == 0)
    def _():
        hg_ref[...] = jnp.zeros_like(hg_ref)

    hg_ref[...] += pg
    hw_ref[...] = jnp.dot(h, wn_ref[...], preferred_element_type=jnp.float32,
                          precision=_PREC)


def _tc_last_body(x_ref, b_ref, a_ref, h_ref, hg_ref):
    h = _prelu(x_ref[...], a_ref[0, 0])
    pg = _onehot_pool(b_ref[...], h)

    @pl.when(pl.program_id(0) == 0)
    def _():
        hg_ref[...] = jnp.zeros_like(hg_ref)

    hg_ref[...] += pg
    h_ref[...] = h


def _row_spec(cols):
    return pl.BlockSpec((RB, cols), lambda i: (i, 0))


def _fix_spec(shape):
    return pl.BlockSpec(shape, lambda i: (0, 0))


_tc_first = pl.pallas_call(
    _tc_first_body,
    grid=(_GRID,),
    in_specs=[_row_spec(IN_FT), _fix_spec((IN_FT, OUT_FT)),
              _fix_spec((OUT_FT, OUT_FT)), _row_spec(128),
              _fix_spec((1, 1))],
    out_specs=[_row_spec(OUT_FT), _fix_spec((NG, OUT_FT))],
    out_shape=[jax.ShapeDtypeStruct((NPAD, OUT_FT), jnp.float32),
               jax.ShapeDtypeStruct((NG, OUT_FT), jnp.float32)],
)

_tc_mid = pl.pallas_call(
    _tc_mid_body,
    grid=(_GRID,),
    in_specs=[_row_spec(OUT_FT), _fix_spec((OUT_FT, OUT_FT)),
              _row_spec(128), _fix_spec((1, 1))],
    out_specs=[_row_spec(OUT_FT), _fix_spec((NG, OUT_FT))],
    out_shape=[jax.ShapeDtypeStruct((NPAD, OUT_FT), jnp.float32),
               jax.ShapeDtypeStruct((NG, OUT_FT), jnp.float32)],
)

_tc_last = pl.pallas_call(
    _tc_last_body,
    grid=(_GRID,),
    in_specs=[_row_spec(OUT_FT), _row_spec(128), _fix_spec((1, 1))],
    out_specs=[_row_spec(OUT_FT), _fix_spec((NG, OUT_FT))],
    out_shape=[jax.ShapeDtypeStruct((NPAD, OUT_FT), jnp.float32),
               jax.ShapeDtypeStruct((NG, OUT_FT), jnp.float32)],
)


def kernel(feat, edge_index, batch_indices, edge_weight, W0, W1, W2,
           a0, a1, a2):
    feat_p = jnp.zeros((NPAD, IN_FT), jnp.float32).at[:N].set(feat)
    batch_p = jnp.full((NPAD,), NG, jnp.int32).at[:N].set(batch_indices)
    batch_b = jnp.broadcast_to(batch_p[:, None], (NPAD, 128))
    src = edge_index[0]
    dst = edge_index[1]

    lists = _filter(src, dst, edge_weight)[0]

    aggf = _agg128(feat_p, lists)[0].reshape(NPAD, IN_FT)
    hw1, hg0 = _tc_first(aggf, W0, W1, batch_b, a0.reshape(1, 1))

    agg1 = _agg256(hw1, lists)[0].reshape(NPAD, OUT_FT)
    hw2, hg1 = _tc_mid(agg1, W2, batch_b, a1.reshape(1, 1))

    agg2 = _agg256(hw2, lists)[0].reshape(NPAD, OUT_FT)
    h3, hg2 = _tc_last(agg2, batch_b, a2.reshape(1, 1))

    return h3[:N], jnp.concatenate([hg0, hg1, hg2], axis=-1)
